# baseline (speedup 1.0000x reference)
import jax
import jax.numpy as jnp
from jax import lax
from jax.experimental import pallas as pl
from jax.experimental.pallas import tpu as pltpu
from jax.experimental.pallas import tpu_sc as plsc

_S = 200
_B = 4096
_D = 64
_C = 128
_N = _S * _B
_NCHUNK = _N // _C
_CPS = _B // _C
_NC = 2
_NS = 16
_NW = _NC * _NS
_PER_W = _NCHUNK // _NW
_L = 16
_NQ = _D // _L


def _emb_body(xp_hbm, x_hbm, table_hbm, pos_hbm, out_hbm,
              idx_v, xv_v, rows_v, outr_v, hm_v, pos_v, gsem, osem):
    wid = lax.axis_index("s") * _NC + lax.axis_index("c")
    pltpu.sync_copy(pos_hbm.at[pl.ds(0, _S)], pos_v)
    lane = lax.iota(jnp.int32, _L)

    def stage_and_fire(t, buf):
        g = wid * _PER_W + t
        pltpu.sync_copy(xp_hbm.at[pl.ds(g, 1)], idx_v.at[buf])
        pltpu.sync_copy(x_hbm.at[pl.ds(g, 1)], xv_v.at[buf])
        pltpu.async_copy(
            table_hbm.at[idx_v.at[buf, 0]], rows_v.at[buf], gsem
        )

    def drain_gather(buf):
        pltpu.make_async_copy(
            table_hbm.at[idx_v.at[buf, 0]], rows_v.at[buf], gsem
        ).wait()

    stage_and_fire(0, 0)

    def chunk_body(t, carry):
        g = wid * _PER_W + t
        buf = t % 2

        @pl.when(t + 1 < _PER_W)
        def _():
            stage_and_fire(t + 1, (t + 1) % 2)

        drain_gather(buf)

        s_idx = g // _CPS
        pos_regs = [pos_v[s_idx, pl.ds(q * _L, _L)] for q in range(_NQ)]

        def hmb(tb16, c2):
            hv = (xv_v[buf, 0, pl.ds(tb16 * _L, _L)] & 1).astype(jnp.float32)
            for j in range(_L):
                plsc.store_scatter(hm_v, [(tb16 * _L + lane) * _L + j], hv)
            return c2

        lax.fori_loop(0, _C // _L, hmb, 0)

        @pl.when(t >= 2)
        def _():
            pltpu.make_async_copy(
                outr_v.at[buf], out_hbm.at[pl.ds((g - 2) * _C, _C)], osem
            ).wait()

        def sel(tt, c2):
            m = hm_v[pl.ds(tt * _L, _L)]
            for q in range(_NQ):
                lo = rows_v[buf, tt, pl.ds(q * _L, _L)]
                hi = rows_v[buf, tt, pl.ds(_D + q * _L, _L)]
                outr_v[buf, tt, pl.ds(q * _L, _L)] = lo + m * (hi - lo) + pos_regs[q]
            return c2

        lax.fori_loop(0, _C, sel, 0, unroll=4)

        pltpu.async_copy(outr_v.at[buf], out_hbm.at[pl.ds(g * _C, _C)], osem)
        return carry

    lax.fori_loop(0, _PER_W, chunk_body, 0)

    for tail in (_PER_W - 2, _PER_W - 1):
        g = wid * _PER_W + tail
        pltpu.make_async_copy(
            outr_v.at[tail % 2], out_hbm.at[pl.ds(g * _C, _C)], osem
        ).wait()


@jax.jit
def _run(x, embed_table, pos_table):
    mesh = plsc.VectorSubcoreMesh(core_axis_name="c", subcore_axis_name="s")
    x2d = x.reshape(_N // _C, _C)
    xp2d = x2d >> 1
    tbl = lax.optimization_barrier(embed_table.reshape(500000, 128))
    out = pl.kernel(
        _emb_body,
        out_type=jax.ShapeDtypeStruct((_N, _D), jnp.float32),
        mesh=mesh,
        scratch_types=[
            pltpu.VMEM((2, 1, _C), jnp.int32),
            pltpu.VMEM((2, 1, _C), jnp.int32),
            pltpu.VMEM((2, _C, 128), jnp.float32),
            pltpu.VMEM((2, _C, _D), jnp.float32),
            pltpu.VMEM((_C * _L,), jnp.float32),
            pltpu.VMEM((_S, _D), jnp.float32),
            pltpu.SemaphoreType.DMA,
            pltpu.SemaphoreType.DMA,
        ],
        compiler_params=pltpu.CompilerParams(
            use_tc_tiling_on_sc=True, needs_layout_passes=False
        ),
    )(xp2d, x2d, tbl, pos_table)
    return out.reshape(_S, _B, _D)


def kernel(x, embed_table, pos_table):
    return _run(x, embed_table, pos_table)


# speedup vs baseline: 1.5391x; 1.5391x over previous
import jax
import jax.numpy as jnp
from jax import lax
from jax.experimental import pallas as pl
from jax.experimental.pallas import tpu as pltpu
from jax.experimental.pallas import tpu_sc as plsc

_S = 200
_B = 4096
_D = 64
_C = 512
_SUB = 128
_NSUB = _C // _SUB
_N = _S * _B
_NCHUNK = _N // _C
_CPS = _B // _C
_NC = 2
_NS = 16
_NW = _NC * _NS
_PER_W = _NCHUNK // _NW
_L = 16
_NQ = _D // _L


def _emb_body(x_hbm, table_hbm, pos_hbm, out_hbm, idx_v, rows_v, pos_v, gsem, osem):
    wid = lax.axis_index("s") * _NC + lax.axis_index("c")
    pltpu.sync_copy(pos_hbm.at[pl.ds(0, _S)], pos_v)

    def stage_and_fire(t, buf):
        g = wid * _PER_W + t
        pltpu.sync_copy(x_hbm.at[pl.ds(g * _NSUB, _NSUB)], idx_v.at[buf])
        for j in range(_NSUB):
            pltpu.async_copy(
                table_hbm.at[idx_v.at[buf, j]],
                rows_v.at[buf, pl.ds(j * _SUB, _SUB)],
                gsem,
            )

    def drain_gather(buf):
        for j in range(_NSUB):
            pltpu.make_async_copy(
                table_hbm.at[idx_v.at[buf, j]],
                rows_v.at[buf, pl.ds(j * _SUB, _SUB)],
                gsem,
            ).wait()

    stage_and_fire(0, 0)

    def chunk_body(t, carry):
        g = wid * _PER_W + t
        buf = t % 2

        @pl.when(t + 1 < _PER_W)
        def _():
            stage_and_fire(t + 1, (t + 1) % 2)

        drain_gather(buf)

        s_idx = g // _CPS
        pos_regs = [pos_v[s_idx, pl.ds(q * _L, _L)] for q in range(_NQ)]

        def row_body(i, c2):
            for q in range(_NQ):
                plsc.addupdate(rows_v.at[buf, i, pl.ds(q * _L, _L)], pos_regs[q])
            return c2

        lax.fori_loop(0, _C, row_body, 0, unroll=8)

        @pl.when(t >= 2)
        def _():
            pltpu.make_async_copy(
                rows_v.at[buf],
                out_hbm.at[pl.ds((g - 2) * _C, _C)],
                osem,
            ).wait()

        pltpu.async_copy(rows_v.at[buf], out_hbm.at[pl.ds(g * _C, _C)], osem)
        return carry

    lax.fori_loop(0, _PER_W, chunk_body, 0)

    for tail in (_PER_W - 2, _PER_W - 1):
        g = wid * _PER_W + tail
        pltpu.make_async_copy(
            rows_v.at[tail % 2],
            out_hbm.at[pl.ds(g * _C, _C)],
            osem,
        ).wait()


@jax.jit
def _run(x, embed_table, pos_table):
    mesh = plsc.VectorSubcoreMesh(core_axis_name="c", subcore_axis_name="s")
    x2d = x.reshape(_N // _SUB, _SUB)
    tbl = lax.optimization_barrier(embed_table.reshape(500000, 128))
    tbl = tbl.reshape(1000000, _D)
    out = pl.kernel(
        _emb_body,
        out_type=jax.ShapeDtypeStruct((_N, _D), jnp.float32),
        mesh=mesh,
        scratch_types=[
            pltpu.VMEM((2, _NSUB, _SUB), jnp.int32),
            pltpu.VMEM((2, _C, _D), jnp.float32),
            pltpu.VMEM((_S, _D), jnp.float32),
            pltpu.SemaphoreType.DMA,
            pltpu.SemaphoreType.DMA,
        ],
        compiler_params=pltpu.CompilerParams(
            use_tc_tiling_on_sc=False, needs_layout_passes=False
        ),
    )(x2d, tbl, pos_table)
    return out.reshape(_S, _B, _D)


def kernel(x, embed_table, pos_table):
    return _run(x, embed_table, pos_table)
